# baseline (speedup 1.0000x reference)
import functools

import jax
import jax.numpy as jnp
from jax import lax
from jax.experimental import pallas as pl
from jax.experimental.pallas import tpu as pltpu
from jax.experimental.pallas import tpu_sc as plsc

N = 10000
E = 320000
D_IN = 128
D_H = 128
D_OUT = 64
D1 = 136
D2 = 64

NC = 2
NS = 16
NW = NC * NS
EPW = E // NW
CHUNK = 80
NCHUNK = 126
NPAD = 10240
ROWS_PER_TILE = NPAD // NS


def _make_sc_aggregate(d, nbuf):
    mesh = plsc.VectorSubcoreMesh(core_axis_name="c", subcore_axis_name="s")

    @functools.partial(
        pl.kernel,
        mesh=mesh,
        out_type=jax.ShapeDtypeStruct((NC, NPAD, d), jnp.float32),
        compiler_params=pltpu.CompilerParams(use_tc_tiling_on_sc=False,
                                             needs_layout_passes=False),
        scratch_types=(
            [pltpu.VMEM((NCHUNK, CHUNK), jnp.int32),
             pltpu.VMEM((NCHUNK, CHUNK), jnp.int32)]
            + [pltpu.VMEM((CHUNK, d), jnp.float32)] * nbuf
            + [pltpu.VMEM_SHARED((NPAD, d), jnp.float32)]
            + [pltpu.SemaphoreType.DMA] * nbuf
        ),
    )
    def agg(feat_hbm, src_hbm, dst_hbm, zeros_hbm, out_hbm,
            src_v, dst_v, *rest):
        bufs = rest[:nbuf]
        acc_sh = rest[nbuf]
        sems = rest[nbuf + 1:]
        c = lax.axis_index("c")
        s = lax.axis_index("s")
        b = c * NS + s
        r0 = s * ROWS_PER_TILE
        pltpu.sync_copy(zeros_hbm.at[pl.ds(r0, ROWS_PER_TILE)],
                        acc_sh.at[pl.ds(r0, ROWS_PER_TILE)])
        pltpu.sync_copy(src_hbm.at[b], src_v)
        pltpu.sync_copy(dst_hbm.at[b], dst_v)
        plsc.subcore_barrier()

        def gather(j, t):
            pltpu.async_copy(feat_hbm.at[src_v.at[j]], bufs[t], sems[t])

        def wait_gather(j, t):
            pltpu.make_async_copy(feat_hbm.at[src_v.at[j]],
                                  bufs[t], sems[t]).wait()

        def scatter(j, t):
            pltpu.sync_copy(bufs[t], acc_sh.at[dst_v.at[j]], add=True)

        for t in range(nbuf):
            gather(t, t)

        def pipe(m, carry):
            for t in range(nbuf):
                j = m * nbuf + t
                wait_gather(j, t)
                scatter(j, t)
                gather(j + nbuf, t)
            return carry

        lax.fori_loop(0, NCHUNK // nbuf - 1, pipe, 0)
        for t in range(nbuf):
            j = NCHUNK - nbuf + t
            wait_gather(j, t)
            scatter(j, t)
        plsc.subcore_barrier()
        pltpu.sync_copy(acc_sh.at[pl.ds(r0, ROWS_PER_TILE)],
                        out_hbm.at[c, pl.ds(r0, ROWS_PER_TILE)])

    return agg


_BN = 1000
_BN1 = 1024


def _tc_forward(p, Wt1, bt1, Wp1, bp1, Wt2, Wp2):
    def body(p_ref, wt1_ref, bt1_ref, wp1_ref, bp1_ref, wt2_ref,
             wp2_ref, z_ref, inv_ref):
        sblk = p_ref[0] + p_ref[1]
        inv = 1.0 / jnp.maximum(sblk[:, D_IN:D_IN + 1], 1.0)
        a = sblk[:, :D_IN] * inv
        ht = jnp.maximum(
            jnp.dot(a, wt1_ref[...], preferred_element_type=jnp.float32)
            + bt1_ref[...], 0.0)
        hp = jnp.maximum(
            jnp.dot(a, wp1_ref[...], preferred_element_type=jnp.float32)
            + bp1_ref[...], 0.0)
        z_ref[...] = (
            jnp.dot(hp, wp2_ref[...], preferred_element_type=jnp.float32)
            - jnp.dot(ht, wt2_ref[...], preferred_element_type=jnp.float32))
        inv_ref[...] = inv

    return pl.pallas_call(
        body,
        grid=(NPAD // _BN1,),
        in_specs=[
            pl.BlockSpec((NC, _BN1, D1), lambda i: (0, i, 0)),
            pl.BlockSpec((D_IN, D_H), lambda i: (0, 0)),
            pl.BlockSpec((1, D_H), lambda i: (0, 0)),
            pl.BlockSpec((D_IN, D_H), lambda i: (0, 0)),
            pl.BlockSpec((1, D_H), lambda i: (0, 0)),
            pl.BlockSpec((D_H, D_OUT), lambda i: (0, 0)),
            pl.BlockSpec((D_H, D_OUT), lambda i: (0, 0)),
        ],
        out_specs=[
            pl.BlockSpec((_BN1, D2), lambda i: (i, 0)),
            pl.BlockSpec((_BN1, 1), lambda i: (i, 0)),
        ],
        out_shape=[
            jax.ShapeDtypeStruct((NPAD, D2), jnp.float32),
            jax.ShapeDtypeStruct((NPAD, 1), jnp.float32),
        ],
    )(p, Wt1, bt1, Wp1, bp1, Wt2, Wp2)


def _tc_loss(q, invd, db):
    grid_n = N // _BN

    def body(q_ref, inv_ref, db_ref, out_ref):
        i = pl.program_id(0)
        diff = (q_ref[0] + q_ref[1]) * inv_ref[...] + db_ref[...]
        ssq = jnp.sum(diff * diff)
        prev = jnp.where(i == 0, 0.0, out_ref[0, 0])
        tot = prev + ssq
        out_ref[0, 0] = jnp.where(i == grid_n - 1,
                                  tot * (1.0 / (N * D_OUT)), tot)

    return pl.pallas_call(
        body,
        grid=(grid_n,),
        in_specs=[
            pl.BlockSpec((NC, _BN, D2), lambda i: (0, i, 0)),
            pl.BlockSpec((_BN, 1), lambda i: (i, 0)),
            pl.BlockSpec((1, D2), lambda i: (0, 0)),
        ],
        out_specs=pl.BlockSpec(memory_space=pltpu.SMEM),
        out_shape=jax.ShapeDtypeStruct((1, 1), jnp.float32),
    )(q, invd, db)


def _partition(e, fill):
    e2 = e.reshape(NW, EPW)
    pad = jnp.full((NW, NCHUNK * CHUNK - EPW), fill, jnp.int32)
    return jnp.concatenate([e2, pad], axis=1).reshape(NW, NCHUNK, CHUNK)


def kernel(x, edge_index, Wt1, bt1, Wt2, bt2, Wp1, bp1, Wp2, bp2):
    src = _partition(edge_index[0], 0)
    dst = _partition(edge_index[1], N)
    xaug = jnp.concatenate(
        [x, jnp.ones((N, 1), jnp.float32),
         jnp.zeros((N, D1 - D_IN - 1), jnp.float32)], axis=1)
    zeros1 = jnp.zeros((NPAD, D1), jnp.float32)
    zeros2 = jnp.zeros((NPAD, D2), jnp.float32)

    p1 = _make_sc_aggregate(D1, 2)(xaug, src, dst, zeros1)
    z, invd = _tc_forward(p1, Wt1, bt1.reshape(1, D_H), Wp1,
                          bp1.reshape(1, D_H), Wt2, Wp2)
    p2 = _make_sc_aggregate(D2, 3)(z, src, dst, zeros2)
    loss = _tc_loss(p2, invd, (bp2 - bt2).reshape(1, D2))
    return loss[0, 0]


# speedup vs baseline: 1.0008x; 1.0008x over previous
import functools

import jax
import jax.numpy as jnp
from jax import lax
from jax.experimental import pallas as pl
from jax.experimental.pallas import tpu as pltpu
from jax.experimental.pallas import tpu_sc as plsc

N = 10000
E = 320000
D_IN = 128
D_H = 128
D_OUT = 64
D1 = 136
D2 = 64

NC = 2
NS = 16
NW = NC * NS
EPW = E // NW
CHUNK = 80
NCHUNK = 126
NPAD = 10240
ROWS_PER_TILE = NPAD // NS


def _make_sc_aggregate(d, nbuf):
    mesh = plsc.VectorSubcoreMesh(core_axis_name="c", subcore_axis_name="s")

    @functools.partial(
        pl.kernel,
        mesh=mesh,
        out_type=jax.ShapeDtypeStruct((NC, NPAD, d), jnp.float32),
        compiler_params=pltpu.CompilerParams(use_tc_tiling_on_sc=False,
                                             needs_layout_passes=False),
        scratch_types=(
            [pltpu.VMEM((NCHUNK, CHUNK), jnp.int32),
             pltpu.VMEM((NCHUNK, CHUNK), jnp.int32)]
            + [pltpu.VMEM((CHUNK, d), jnp.float32)] * nbuf
            + [pltpu.VMEM_SHARED((NPAD, d), jnp.float32)]
            + [pltpu.SemaphoreType.DMA] * nbuf
        ),
    )
    def agg(feat_hbm, src_hbm, dst_hbm, zeros_hbm, out_hbm,
            src_v, dst_v, *rest):
        bufs = rest[:nbuf]
        acc_sh = rest[nbuf]
        sems = rest[nbuf + 1:]
        c = lax.axis_index("c")
        s = lax.axis_index("s")
        b = c * NS + s
        r0 = s * ROWS_PER_TILE
        pltpu.sync_copy(zeros_hbm.at[pl.ds(r0, ROWS_PER_TILE)],
                        acc_sh.at[pl.ds(r0, ROWS_PER_TILE)])
        pltpu.sync_copy(src_hbm.at[b], src_v)
        pltpu.sync_copy(dst_hbm.at[b], dst_v)
        plsc.subcore_barrier()

        def gather(j, t):
            pltpu.async_copy(feat_hbm.at[src_v.at[j]], bufs[t], sems[t])

        def wait_gather(j, t):
            pltpu.make_async_copy(feat_hbm.at[src_v.at[j]],
                                  bufs[t], sems[t]).wait()

        def scatter(j, t):
            pltpu.sync_copy(bufs[t], acc_sh.at[dst_v.at[j]], add=True)

        for t in range(nbuf):
            gather(t, t)

        def pipe(m, carry):
            for t in range(nbuf):
                j = m * nbuf + t
                wait_gather(j, t)
                scatter(j, t)
                gather(j + nbuf, t)
            return carry

        lax.fori_loop(0, NCHUNK // nbuf - 1, pipe, 0)
        for t in range(nbuf):
            j = NCHUNK - nbuf + t
            wait_gather(j, t)
            scatter(j, t)
        plsc.subcore_barrier()
        pltpu.sync_copy(acc_sh.at[pl.ds(r0, ROWS_PER_TILE)],
                        out_hbm.at[c, pl.ds(r0, ROWS_PER_TILE)])

    return agg


_BN = 1000
_BN1 = 1024


def _tc_forward(p, Wt1, bt1, Wp1, bp1, Wt2, Wp2):
    def body(p_ref, wt1_ref, bt1_ref, wp1_ref, bp1_ref, wt2_ref,
             wp2_ref, z_ref, inv_ref):
        sblk = p_ref[0] + p_ref[1]
        inv = 1.0 / jnp.maximum(sblk[:, D_IN:D_IN + 1], 1.0)
        a = sblk[:, :D_IN] * inv
        ht = jnp.maximum(
            jnp.dot(a, wt1_ref[...], preferred_element_type=jnp.float32)
            + bt1_ref[...], 0.0)
        hp = jnp.maximum(
            jnp.dot(a, wp1_ref[...], preferred_element_type=jnp.float32)
            + bp1_ref[...], 0.0)
        z_ref[...] = (
            jnp.dot(hp, wp2_ref[...], preferred_element_type=jnp.float32)
            - jnp.dot(ht, wt2_ref[...], preferred_element_type=jnp.float32))
        inv_ref[...] = inv

    return pl.pallas_call(
        body,
        grid=(NPAD // _BN1,),
        in_specs=[
            pl.BlockSpec((NC, _BN1, D1), lambda i: (0, i, 0)),
            pl.BlockSpec((D_IN, D_H), lambda i: (0, 0)),
            pl.BlockSpec((1, D_H), lambda i: (0, 0)),
            pl.BlockSpec((D_IN, D_H), lambda i: (0, 0)),
            pl.BlockSpec((1, D_H), lambda i: (0, 0)),
            pl.BlockSpec((D_H, D_OUT), lambda i: (0, 0)),
            pl.BlockSpec((D_H, D_OUT), lambda i: (0, 0)),
        ],
        out_specs=[
            pl.BlockSpec((_BN1, D2), lambda i: (i, 0)),
            pl.BlockSpec((_BN1, 1), lambda i: (i, 0)),
        ],
        out_shape=[
            jax.ShapeDtypeStruct((NPAD, D2), jnp.float32),
            jax.ShapeDtypeStruct((NPAD, 1), jnp.float32),
        ],
    )(p, Wt1, bt1, Wp1, bp1, Wt2, Wp2)


def _tc_loss(q, invd, db):
    grid_n = N // _BN

    def body(q_ref, inv_ref, db_ref, out_ref):
        i = pl.program_id(0)
        diff = (q_ref[0] + q_ref[1]) * inv_ref[...] + db_ref[...]
        ssq = jnp.sum(diff * diff)
        prev = jnp.where(i == 0, 0.0, out_ref[0, 0])
        tot = prev + ssq
        out_ref[0, 0] = jnp.where(i == grid_n - 1,
                                  tot * (1.0 / (N * D_OUT)), tot)

    return pl.pallas_call(
        body,
        grid=(grid_n,),
        in_specs=[
            pl.BlockSpec((NC, _BN, D2), lambda i: (0, i, 0)),
            pl.BlockSpec((_BN, 1), lambda i: (i, 0)),
            pl.BlockSpec((1, D2), lambda i: (0, 0)),
        ],
        out_specs=pl.BlockSpec(memory_space=pltpu.SMEM),
        out_shape=jax.ShapeDtypeStruct((1, 1), jnp.float32),
    )(q, invd, db)


def _partition(e, spread_fill):
    e2 = e.reshape(NW, EPW)
    npad_e = NCHUNK * CHUNK - EPW
    if spread_fill:
        pad = jnp.broadcast_to(
            N + jnp.arange(npad_e, dtype=jnp.int32) % (NPAD - N),
            (NW, npad_e))
    else:
        pad = jnp.zeros((NW, npad_e), jnp.int32)
    return jnp.concatenate([e2, pad], axis=1).reshape(NW, NCHUNK, CHUNK)


def kernel(x, edge_index, Wt1, bt1, Wt2, bt2, Wp1, bp1, Wp2, bp2):
    src = _partition(edge_index[0], False)
    dst = _partition(edge_index[1], True)
    xaug = jnp.concatenate(
        [x, jnp.ones((N, 1), jnp.float32),
         jnp.zeros((N, D1 - D_IN - 1), jnp.float32)], axis=1)
    zeros1 = jnp.zeros((NPAD, D1), jnp.float32)
    zeros2 = jnp.zeros((NPAD, D2), jnp.float32)

    p1 = _make_sc_aggregate(D1, 2)(xaug, src, dst, zeros1)
    z, invd = _tc_forward(p1, Wt1, bt1.reshape(1, D_H), Wp1,
                          bp1.reshape(1, D_H), Wt2, Wp2)
    p2 = _make_sc_aggregate(D2, 3)(z, src, dst, zeros2)
    loss = _tc_loss(p2, invd, (bp2 - bt2).reshape(1, D2))
    return loss[0, 0]


# speedup vs baseline: 1.3896x; 1.3885x over previous
import functools

import jax
import jax.numpy as jnp
from jax import lax
from jax.experimental import pallas as pl
from jax.experimental.pallas import tpu as pltpu
from jax.experimental.pallas import tpu_sc as plsc

N = 10000
E = 320000
D_IN = 128
D_H = 128
D_OUT = 64
D1 = 136
D2 = 64

NC = 2
NS = 16
NW = NC * NS
EPW = E // NW
CHUNK = 80
NCHUNK = 125
NPAD = 10240
ROWS_PER_TILE = NPAD // NS


def _make_sc_aggregate(d):
    mesh = plsc.VectorSubcoreMesh(core_axis_name="c", subcore_axis_name="s")

    @functools.partial(
        pl.kernel,
        mesh=mesh,
        out_type=jax.ShapeDtypeStruct((NC, NPAD, d), jnp.float32),
        compiler_params=pltpu.CompilerParams(use_tc_tiling_on_sc=False,
                                             needs_layout_passes=False),
        scratch_types=[
            pltpu.VMEM((NCHUNK, CHUNK), jnp.int32),
            pltpu.VMEM((NCHUNK, CHUNK), jnp.int32),
            pltpu.VMEM((CHUNK, d), jnp.float32),
            pltpu.VMEM((CHUNK, d), jnp.float32),
            pltpu.VMEM_SHARED((NPAD, d), jnp.float32),
            pltpu.SemaphoreType.DMA,
            pltpu.SemaphoreType.DMA,
        ],
    )
    def agg(feat_hbm, src_hbm, dst_hbm, zeros_hbm, out_hbm,
            src_v, dst_v, buf0, buf1, acc_sh, sem0, sem1):
        c = lax.axis_index("c")
        s = lax.axis_index("s")
        b = c * NS + s
        r0 = s * ROWS_PER_TILE
        pltpu.sync_copy(zeros_hbm.at[pl.ds(r0, ROWS_PER_TILE)],
                        acc_sh.at[pl.ds(r0, ROWS_PER_TILE)])
        pltpu.sync_copy(src_hbm.at[b], src_v)
        pltpu.sync_copy(dst_hbm.at[b], dst_v)
        plsc.subcore_barrier()

        def gather(j, buf, sem):
            return pltpu.async_copy(feat_hbm.at[src_v.at[j]], buf, sem)

        def scatter(j, buf):
            pltpu.sync_copy(buf, acc_sh.at[dst_v.at[j]], add=True)

        def wait0(j):
            pltpu.make_async_copy(feat_hbm.at[src_v.at[j]],
                                  buf0, sem0).wait()

        gather(0, buf0, sem0)

        def pipe(k, carry):
            jo = 2 * k + 1
            je = 2 * k + 2
            h_odd = gather(jo, buf1, sem1)
            wait0(je - 2)
            scatter(je - 2, buf0)
            gather(je, buf0, sem0)
            h_odd.wait()
            scatter(jo, buf1)
            return carry

        lax.fori_loop(0, (NCHUNK - 1) // 2, pipe, 0)
        wait0(NCHUNK - 1)
        scatter(NCHUNK - 1, buf0)
        plsc.subcore_barrier()
        pltpu.sync_copy(acc_sh.at[pl.ds(r0, ROWS_PER_TILE)],
                        out_hbm.at[c, pl.ds(r0, ROWS_PER_TILE)])

    return agg


_BN = 1000
_BN1 = 1024


def _tc_forward(p, Wt1, bt1, Wp1, bp1, Wt2, Wp2):
    def body(p_ref, wt1_ref, bt1_ref, wp1_ref, bp1_ref, wt2_ref,
             wp2_ref, z_ref, inv_ref):
        sblk = p_ref[0] + p_ref[1]
        inv = 1.0 / jnp.maximum(sblk[:, D_IN:D_IN + 1], 1.0)
        a = sblk[:, :D_IN] * inv
        ht = jnp.maximum(
            jnp.dot(a, wt1_ref[...], preferred_element_type=jnp.float32)
            + bt1_ref[...], 0.0)
        hp = jnp.maximum(
            jnp.dot(a, wp1_ref[...], preferred_element_type=jnp.float32)
            + bp1_ref[...], 0.0)
        z_ref[...] = (
            jnp.dot(hp, wp2_ref[...], preferred_element_type=jnp.float32)
            - jnp.dot(ht, wt2_ref[...], preferred_element_type=jnp.float32))
        inv_ref[...] = inv

    return pl.pallas_call(
        body,
        grid=(NPAD // _BN1,),
        in_specs=[
            pl.BlockSpec((NC, _BN1, D1), lambda i: (0, i, 0)),
            pl.BlockSpec((D_IN, D_H), lambda i: (0, 0)),
            pl.BlockSpec((1, D_H), lambda i: (0, 0)),
            pl.BlockSpec((D_IN, D_H), lambda i: (0, 0)),
            pl.BlockSpec((1, D_H), lambda i: (0, 0)),
            pl.BlockSpec((D_H, D_OUT), lambda i: (0, 0)),
            pl.BlockSpec((D_H, D_OUT), lambda i: (0, 0)),
        ],
        out_specs=[
            pl.BlockSpec((_BN1, D2), lambda i: (i, 0)),
            pl.BlockSpec((_BN1, 1), lambda i: (i, 0)),
        ],
        out_shape=[
            jax.ShapeDtypeStruct((NPAD, D2), jnp.float32),
            jax.ShapeDtypeStruct((NPAD, 1), jnp.float32),
        ],
    )(p, Wt1, bt1, Wp1, bp1, Wt2, Wp2)


def _tc_loss(q, invd, db):
    grid_n = N // _BN

    def body(q_ref, inv_ref, db_ref, out_ref):
        i = pl.program_id(0)
        diff = (q_ref[0] + q_ref[1]) * inv_ref[...] + db_ref[...]
        ssq = jnp.sum(diff * diff)
        prev = jnp.where(i == 0, 0.0, out_ref[0, 0])
        tot = prev + ssq
        out_ref[0, 0] = jnp.where(i == grid_n - 1,
                                  tot * (1.0 / (N * D_OUT)), tot)

    return pl.pallas_call(
        body,
        grid=(grid_n,),
        in_specs=[
            pl.BlockSpec((NC, _BN, D2), lambda i: (0, i, 0)),
            pl.BlockSpec((_BN, 1), lambda i: (i, 0)),
            pl.BlockSpec((1, D2), lambda i: (0, 0)),
        ],
        out_specs=pl.BlockSpec(memory_space=pltpu.SMEM),
        out_shape=jax.ShapeDtypeStruct((1, 1), jnp.float32),
    )(q, invd, db)


def kernel(x, edge_index, Wt1, bt1, Wt2, bt2, Wp1, bp1, Wp2, bp2):
    src = edge_index[0].reshape(NW, NCHUNK, CHUNK)
    dst = edge_index[1].reshape(NW, NCHUNK, CHUNK)
    xaug = jnp.concatenate(
        [x, jnp.ones((N, 1), jnp.float32),
         jnp.zeros((N, D1 - D_IN - 1), jnp.float32)], axis=1)
    zeros1 = jnp.zeros((NPAD, D1), jnp.float32)
    zeros2 = jnp.zeros((NPAD, D2), jnp.float32)

    p1 = _make_sc_aggregate(D1)(xaug, src, dst, zeros1)
    z, invd = _tc_forward(p1, Wt1, bt1.reshape(1, D_H), Wp1,
                          bp1.reshape(1, D_H), Wt2, Wp2)
    p2 = _make_sc_aggregate(D2)(z, src, dst, zeros2)
    loss = _tc_loss(p2, invd, (bp2 - bt2).reshape(1, D2))
    return loss[0, 0]
